# baseline (speedup 1.0000x reference)
import functools

import jax
import jax.numpy as jnp
from jax import lax
from jax.experimental import pallas as pl
from jax.experimental.pallas import tpu as pltpu
from jax.experimental.pallas import tpu_sc as plsc

N = 10000
F = 128
NC = 2
NS = 16
NW = NC * NS
CHUNK = 128
N_PAD = 10112
RPT = N_PAD // NS
RB = 1000
GRID = N // RB

_E = 320000
NBUF = 3
CPW0 = 123
CPW1 = 34
E_PAD = NS * (CPW0 + CPW1) * CHUNK
CPWS = 2 * (-(-_E // (NW * CHUNK * 2)))
E_PADS = NW * CPWS * CHUNK

_mesh = plsc.VectorSubcoreMesh(core_axis_name="c", subcore_axis_name="s")


def _zero_tile(buf):
    def zrow(i, _):
        for j in range(F // 16):
            buf[i, pl.ds(j * 16, 16)] = jnp.zeros((16,), jnp.float32)
        return 0

    lax.fori_loop(0, CHUNK, zrow, 0)


def _deg_body(idx_hbm, out_hbm, idx0, idx1, ones_v, acc_sh, isem0, isem1):
    c = lax.axis_index("c")
    s = lax.axis_index("s")
    w = c * NS + s
    idxb = (idx0, idx1)
    isems = (isem0, isem1)

    _zero_tile(ones_v)
    base = s * RPT
    for k in range(RPT // CHUNK):
        pltpu.sync_copy(ones_v, acc_sh.at[pl.ds(base + k * CHUNK, CHUNK)])
    if RPT % CHUNK:
        pltpu.sync_copy(ones_v.at[pl.ds(0, RPT % CHUNK)],
                        acc_sh.at[pl.ds(base + (RPT // CHUNK) * CHUNK,
                                        RPT % CHUNK)])
    plsc.subcore_barrier()

    def frow(i, _):
        for j in range(F // 16):
            ones_v[i, pl.ds(j * 16, 16)] = jnp.full((16,), 1.0, jnp.float32)
        return 0

    lax.fori_loop(0, CHUNK, frow, 0)

    def istart(j, b):
        pltpu.async_copy(idx_hbm.at[w, j], idxb[b], isems[b])

    def iwait(b):
        pltpu.make_async_copy(idx_hbm.at[w, 0], idxb[b], isems[b]).wait()

    istart(0, 0)
    istart(1, 1)

    def chunk_step(j, b):
        iwait(b)
        pltpu.sync_copy(ones_v, acc_sh.at[idxb[b].at[0]], add=True)

        @pl.when(j + 2 < CPWS)
        def _():
            istart(j + 2, b)

    def pair(i, _):
        chunk_step(2 * i, 0)
        chunk_step(2 * i + 1, 1)
        return 0

    lax.fori_loop(0, CPWS // 2, pair, 0)
    plsc.subcore_barrier()
    pltpu.sync_copy(acc_sh.at[pl.ds(base, RPT)], out_hbm.at[c, pl.ds(base, RPT)])


def _sc_degrees(idx4):
    return pl.kernel(
        _deg_body,
        out_type=jax.ShapeDtypeStruct((NC, N_PAD, F), jnp.float32),
        mesh=_mesh,
        scratch_types=[
            pltpu.VMEM((1, CHUNK), jnp.int32),
            pltpu.VMEM((1, CHUNK), jnp.int32),
            pltpu.VMEM((CHUNK, F), jnp.float32),
            pltpu.VMEM_SHARED((N_PAD, F), jnp.float32),
            pltpu.SemaphoreType.DMA,
            pltpu.SemaphoreType.DMA,
        ],
    )(idx4)


def _agg_body(y_hbm, idx_hbm, out_hbm,
              idx0, idx1, idx2, rows0, rows1, rows2, acc_sh,
              isem0, isem1, isem2, gsem0, gsem1, gsem2):
    c = lax.axis_index("c")
    s = lax.axis_index("s")
    w = c * NS + s
    idxb = (idx0, idx1, idx2)
    rowsb = (rows0, rows1, rows2)
    isems = (isem0, isem1, isem2)
    gsems = (gsem0, gsem1, gsem2)

    _zero_tile(rows0)
    base = s * RPT
    for k in range(RPT // CHUNK):
        pltpu.sync_copy(rows0, acc_sh.at[pl.ds(base + k * CHUNK, CHUNK)])
    if RPT % CHUNK:
        pltpu.sync_copy(rows0.at[pl.ds(0, RPT % CHUNK)],
                        acc_sh.at[pl.ds(base + (RPT // CHUNK) * CHUNK,
                                        RPT % CHUNK)])
    plsc.subcore_barrier()

    def istart(j, b):
        pltpu.async_copy(idx_hbm.at[w, j], idxb[b], isems[b])

    def iwait(b):
        pltpu.make_async_copy(idx_hbm.at[w, 0], idxb[b], isems[b]).wait()

    def gstart(b):
        pltpu.async_copy(y_hbm.at[idxb[b].at[0]], rowsb[b], gsems[b])

    def gwait(b):
        pltpu.make_async_copy(y_hbm.at[idxb[b].at[0]], rowsb[b], gsems[b]).wait()

    @pl.when(c == 0)
    def _ring():
        for k in range(NBUF):
            istart(k, k)
        for k in range(NBUF - 1):
            iwait(k)
            gstart(k)

        def chunk_step(j, b):
            gwait(b)

            @pl.when(j + NBUF - 1 < CPW0)
            def _():
                iwait((b + NBUF - 1) % NBUF)
                gstart((b + NBUF - 1) % NBUF)

            pltpu.sync_copy(rowsb[b], acc_sh.at[idxb[b].at[1]], add=True)

            @pl.when(j + NBUF < CPW0)
            def _():
                istart(j + NBUF, b)

        def group(i, _):
            for k in range(NBUF):
                chunk_step(NBUF * i + k, k)
            return 0

        lax.fori_loop(0, CPW0 // NBUF, group, 0)

    @pl.when(c == 1)
    def _seq():
        def body(j, _):
            istart(j, 0)
            iwait(0)
            gstart(0)
            gwait(0)
            pltpu.sync_copy(rowsb[0], acc_sh.at[idxb[0].at[1]], add=True)
            return 0

        lax.fori_loop(0, CPW1, body, 0)
    plsc.subcore_barrier()
    pltpu.sync_copy(acc_sh.at[pl.ds(base, RPT)], out_hbm.at[c, pl.ds(base, RPT)])


def _sc_aggregate(y, idx4):
    return pl.kernel(
        _agg_body,
        out_type=jax.ShapeDtypeStruct((NC, N_PAD, F), jnp.float32),
        mesh=_mesh,
        scratch_types=(
            [pltpu.VMEM((2, CHUNK), jnp.int32)] * NBUF
            + [pltpu.VMEM((CHUNK, F), jnp.float32)] * NBUF
            + [pltpu.VMEM_SHARED((N_PAD, F), jnp.float32)]
            + [pltpu.SemaphoreType.DMA] * (2 * NBUF)
        ),
    )(y, idx4)


def _bn_block(t, st, g, bt):
    inv_n = 1.0 / N
    mean = st[0:1, :] * inv_n
    var = st[1:2, :] * inv_n - mean * mean
    istd = lax.rsqrt(var + 1e-5)
    return (t - mean) * istd * g + bt


_spec_p0 = pl.BlockSpec((1, RB, F), lambda i: (0, i, 0))
_spec_p1 = pl.BlockSpec((1, RB, F), lambda i: (1, i, 0))
_spec_d0 = pl.BlockSpec((1, RB, 16), lambda i: (0, i, 0))
_spec_d1 = pl.BlockSpec((1, RB, 16), lambda i: (1, i, 0))
_spec_row = pl.BlockSpec((RB, F), lambda i: (i, 0))
_spec_w = pl.BlockSpec((F, F), lambda i: (0, 0))
_spec_vec = pl.BlockSpec((1, F), lambda i: (0, 0))
_spec_st = pl.BlockSpec((8, F), lambda i: (0, 0))
_spec_dis = pl.BlockSpec((RB, 1), lambda i: (i, 0))


def _mm1_dis_body(d0_ref, d1_ref, x_ref, w_ref, dis_ref, y_ref):
    dis = lax.rsqrt(1.0 + d0_ref[0][:, :1] + d1_ref[0][:, :1])
    dis_ref[...] = dis
    y_ref[...] = dis * jnp.dot(x_ref[...], w_ref[...],
                               preferred_element_type=jnp.float32)


def _tc_mm1_dis(degp, x, W):
    return pl.pallas_call(
        _mm1_dis_body,
        grid=(GRID,),
        in_specs=[_spec_p0, _spec_p1, _spec_row, _spec_w],
        out_specs=[_spec_dis, _spec_row],
        out_shape=[jax.ShapeDtypeStruct((N, 1), jnp.float32),
                   jax.ShapeDtypeStruct((N, F), jnp.float32)],
    )(degp, degp, x, W)


def _mm_bn_body(t_ref, st_ref, g_ref, bt_ref, w_ref, dis_ref, y_ref):
    h = _bn_block(t_ref[...], st_ref[...], g_ref[...], bt_ref[...])
    y_ref[...] = dis_ref[...] * jnp.dot(h, w_ref[...],
                                        preferred_element_type=jnp.float32)


def _tc_matmul_bn(t, st, g, bt, W, dis):
    return pl.pallas_call(
        _mm_bn_body,
        grid=(GRID,),
        in_specs=[_spec_row, _spec_st, _spec_vec, _spec_vec, _spec_w,
                  _spec_dis],
        out_specs=_spec_row,
        out_shape=jax.ShapeDtypeStruct((N, F), jnp.float32),
    )(t, st, g, bt, W, dis)


def _mm3_body(x_ref, t1_ref, st1_ref, t2_ref, st2_ref, g_ref, bt_ref,
              w0_ref, w1_ref, w2_ref, dis_ref, y_ref):
    h1 = _bn_block(t1_ref[...], st1_ref[...], g_ref[...], bt_ref[...])
    h2 = _bn_block(t2_ref[...], st2_ref[...], g_ref[...], bt_ref[...])
    acc = jnp.dot(x_ref[...], w0_ref[...], preferred_element_type=jnp.float32)
    acc += jnp.dot(h1, w1_ref[...], preferred_element_type=jnp.float32)
    acc += jnp.dot(h2, w2_ref[...], preferred_element_type=jnp.float32)
    y_ref[...] = dis_ref[...] * acc


def _tc_matmul3_bn(x, t1, st1, t2, st2, g, bt, w0, w1, w2, dis):
    return pl.pallas_call(
        _mm3_body,
        grid=(GRID,),
        in_specs=[_spec_row, _spec_row, _spec_st, _spec_row, _spec_st,
                  _spec_vec, _spec_vec, _spec_w, _spec_w, _spec_w, _spec_dis],
        out_specs=_spec_row,
        out_shape=jax.ShapeDtypeStruct((N, F), jnp.float32),
    )(x, t1, st1, t2, st2, g, bt, w0, w1, w2, dis)


def _post_body(p0_ref, p1_ref, y_ref, dis_ref, b_ref, t_ref, st_ref):
    i = pl.program_id(0)
    pre = dis_ref[...] * (p0_ref[0] + p1_ref[0] + y_ref[...]) + b_ref[...]
    t = jnp.maximum(pre, 0.0)
    t_ref[...] = t
    ssum = jnp.sum(t, axis=0, keepdims=True)
    ssq = jnp.sum(t * t, axis=0, keepdims=True)
    st = jnp.concatenate([ssum, ssq, jnp.zeros((6, F), jnp.float32)], axis=0)

    @pl.when(i == 0)
    def _():
        st_ref[...] = st

    @pl.when(i != 0)
    def _():
        st_ref[...] += st


def _tc_post(p, y, dis, b):
    return pl.pallas_call(
        _post_body,
        grid=(GRID,),
        in_specs=[_spec_p0, _spec_p1, _spec_row, _spec_dis, _spec_vec],
        out_specs=[_spec_row, _spec_st],
        out_shape=[jax.ShapeDtypeStruct((N, F), jnp.float32),
                   jax.ShapeDtypeStruct((8, F), jnp.float32)],
    )(p, p, y, dis, b)


def _final_body(p0_ref, p1_ref, y_ref, dis_ref, b_ref, o_ref):
    pre = dis_ref[...] * (p0_ref[0] + p1_ref[0] + y_ref[...]) + b_ref[...]
    o_ref[...] = jnp.maximum(pre, 0.0)


def _tc_final(p, y, dis, b):
    return pl.pallas_call(
        _final_body,
        grid=(GRID,),
        in_specs=[_spec_p0, _spec_p1, _spec_row, _spec_dis, _spec_vec],
        out_specs=_spec_row,
        out_shape=jax.ShapeDtypeStruct((N, F), jnp.float32),
    )(p, p, y, dis, b)


def kernel(x, edge_index, W1, b1, W2, b2, gamma, beta, W_out, b_out):
    src = edge_index[0].astype(jnp.int32)
    dst = edge_index[1].astype(jnp.int32)
    e = src.shape[0]

    pad = E_PAD - e
    src_p = jnp.concatenate([src, jnp.zeros((pad,), jnp.int32)])
    dst_p = jnp.concatenate([dst, jnp.full((pad,), N, jnp.int32)])
    e0 = NS * CPW0 * CHUNK
    core0 = jnp.concatenate(
        [src_p[:e0].reshape(NS, CPW0, 1, CHUNK),
         dst_p[:e0].reshape(NS, CPW0, 1, CHUNK)], axis=2)
    core1 = jnp.concatenate(
        [src_p[e0:].reshape(NS, CPW1, 1, CHUNK),
         dst_p[e0:].reshape(NS, CPW1, 1, CHUNK)], axis=2)
    core1 = jnp.pad(core1, ((0, 0), (0, CPW0 - CPW1), (0, 0), (0, 0)))
    idx4 = jnp.concatenate([core0, core1], axis=0)

    pads = E_PADS - e
    dst_s = jnp.concatenate([dst, jnp.full((pads,), N, jnp.int32)])
    dst4 = dst_s.reshape(NW, CPWS, 1, CHUNK)

    b1r = b1.reshape(1, F)
    b2r = b2.reshape(1, F)
    bor = b_out.reshape(1, F)
    gr = gamma.reshape(1, F)
    btr = beta.reshape(1, F)

    degp = _sc_degrees(dst4)
    dis, y1 = _tc_mm1_dis(degp, x, W1)

    p1 = _sc_aggregate(y1, idx4)
    t1, st1 = _tc_post(p1, y1, dis, b1r)

    y2 = _tc_matmul_bn(t1, st1, gr, btr, W2, dis)
    p2 = _sc_aggregate(y2, idx4)
    t2, st2 = _tc_post(p2, y2, dis, b2r)

    y3 = _tc_matmul3_bn(x, t1, st1, t2, st2, gr, btr,
                        W_out[:F], W_out[F:2 * F], W_out[2 * F:], dis)
    p3 = _sc_aggregate(y3, idx4)
    return _tc_final(p3, y3, dis, bor)


# speedup vs baseline: 1.0246x; 1.0246x over previous
import functools

import jax
import jax.numpy as jnp
from jax import lax
from jax.experimental import pallas as pl
from jax.experimental.pallas import tpu as pltpu
from jax.experimental.pallas import tpu_sc as plsc

N = 10000
F = 128
NC = 2
NS = 16
NW = NC * NS
CHUNK = 128
N_PAD = 10112
RPT = N_PAD // NS
RB = 1000
GRID = N // RB

_E = 320000
NBUF = 3
CPW0 = 123
CPW1 = 34
E_PAD = NS * (CPW0 + CPW1) * CHUNK
CPWS = 2 * (-(-_E // (NW * CHUNK * 2)))
E_PADS = NW * CPWS * CHUNK

_mesh = plsc.VectorSubcoreMesh(core_axis_name="c", subcore_axis_name="s")


def _zero_tile(buf):
    def zrow(i, _):
        for j in range(F // 16):
            buf[i, pl.ds(j * 16, 16)] = jnp.zeros((16,), jnp.float32)
        return 0

    lax.fori_loop(0, CHUNK, zrow, 0)


def _deg_body(idx_hbm, out_hbm, idx0, idx1, ones_v, acc_sh, isem0, isem1):
    c = lax.axis_index("c")
    s = lax.axis_index("s")
    w = c * NS + s
    idxb = (idx0, idx1)
    isems = (isem0, isem1)

    _zero_tile(ones_v)
    base = s * RPT
    for k in range(RPT // CHUNK):
        pltpu.sync_copy(ones_v, acc_sh.at[pl.ds(base + k * CHUNK, CHUNK)])
    if RPT % CHUNK:
        pltpu.sync_copy(ones_v.at[pl.ds(0, RPT % CHUNK)],
                        acc_sh.at[pl.ds(base + (RPT // CHUNK) * CHUNK,
                                        RPT % CHUNK)])
    plsc.subcore_barrier()

    def frow(i, _):
        for j in range(F // 16):
            ones_v[i, pl.ds(j * 16, 16)] = jnp.full((16,), 1.0, jnp.float32)
        return 0

    lax.fori_loop(0, CHUNK, frow, 0)

    def istart(j, b):
        pltpu.async_copy(idx_hbm.at[w, j], idxb[b], isems[b])

    def iwait(b):
        pltpu.make_async_copy(idx_hbm.at[w, 0], idxb[b], isems[b]).wait()

    istart(0, 0)
    istart(1, 1)

    def chunk_step(j, b):
        iwait(b)
        pltpu.sync_copy(ones_v, acc_sh.at[idxb[b].at[0]], add=True)

        @pl.when(j + 2 < CPWS)
        def _():
            istart(j + 2, b)

    def pair(i, _):
        chunk_step(2 * i, 0)
        chunk_step(2 * i + 1, 1)
        return 0

    lax.fori_loop(0, CPWS // 2, pair, 0)
    plsc.subcore_barrier()
    pltpu.sync_copy(acc_sh.at[pl.ds(base, RPT)], out_hbm.at[c, pl.ds(base, RPT)])


def _sc_degrees(idx4):
    return pl.kernel(
        _deg_body,
        out_type=jax.ShapeDtypeStruct((NC, N_PAD, F), jnp.float32),
        mesh=_mesh,
        scratch_types=[
            pltpu.VMEM((1, CHUNK), jnp.int32),
            pltpu.VMEM((1, CHUNK), jnp.int32),
            pltpu.VMEM((CHUNK, F), jnp.float32),
            pltpu.VMEM_SHARED((N_PAD, F), jnp.float32),
            pltpu.SemaphoreType.DMA,
            pltpu.SemaphoreType.DMA,
        ],
    )(idx4)


def _agg_body(y_hbm, idx_hbm, out_hbm,
              idx0, idx1, idx2, rows0, rows1, rows2, acc_sh,
              isem0, isem1, isem2, gsem0, gsem1, gsem2):
    c = lax.axis_index("c")
    s = lax.axis_index("s")
    w = c * NS + s
    idxb = (idx0, idx1, idx2)
    rowsb = (rows0, rows1, rows2)
    isems = (isem0, isem1, isem2)
    gsems = (gsem0, gsem1, gsem2)

    _zero_tile(rows0)
    base = s * RPT
    for k in range(RPT // CHUNK):
        pltpu.sync_copy(rows0, acc_sh.at[pl.ds(base + k * CHUNK, CHUNK)])
    if RPT % CHUNK:
        pltpu.sync_copy(rows0.at[pl.ds(0, RPT % CHUNK)],
                        acc_sh.at[pl.ds(base + (RPT // CHUNK) * CHUNK,
                                        RPT % CHUNK)])
    plsc.subcore_barrier()

    def istart(j, b):
        pltpu.async_copy(idx_hbm.at[w, j], idxb[b], isems[b])

    def iwait(b):
        pltpu.make_async_copy(idx_hbm.at[w, 0], idxb[b], isems[b]).wait()

    def gstart(b):
        pltpu.async_copy(y_hbm.at[idxb[b].at[0]], rowsb[b], gsems[b])

    def gwait(b):
        pltpu.make_async_copy(y_hbm.at[idxb[b].at[0]], rowsb[b], gsems[b]).wait()

    @pl.when(c == 0)
    def _ring():
        for k in range(NBUF):
            istart(k, k)
        for k in range(NBUF - 1):
            iwait(k)
            gstart(k)

        def chunk_step(j, b):
            gwait(b)

            @pl.when(j + NBUF - 1 < CPW0)
            def _():
                iwait((b + NBUF - 1) % NBUF)
                gstart((b + NBUF - 1) % NBUF)

            pltpu.sync_copy(rowsb[b], acc_sh.at[idxb[b].at[1]], add=True)

            @pl.when(j + NBUF < CPW0)
            def _():
                istart(j + NBUF, b)

        def group(i, _):
            for k in range(NBUF):
                chunk_step(NBUF * i + k, k)
            return 0

        lax.fori_loop(0, CPW0 // NBUF, group, 0)

    @pl.when(c == 1)
    def _seq():
        istart(0, 0)
        istart(1, 1)

        def cstep(j, b):
            iwait(b)
            gstart(b)
            gwait(b)
            pltpu.sync_copy(rowsb[b], acc_sh.at[idxb[b].at[1]], add=True)

            @pl.when(j + 2 < CPW1)
            def _():
                istart(j + 2, b)

        def pair(i, _):
            cstep(2 * i, 0)
            cstep(2 * i + 1, 1)
            return 0

        lax.fori_loop(0, CPW1 // 2, pair, 0)
    plsc.subcore_barrier()
    pltpu.sync_copy(acc_sh.at[pl.ds(base, RPT)], out_hbm.at[c, pl.ds(base, RPT)])


def _sc_aggregate(y, idx4):
    return pl.kernel(
        _agg_body,
        out_type=jax.ShapeDtypeStruct((NC, N_PAD, F), jnp.float32),
        mesh=_mesh,
        scratch_types=(
            [pltpu.VMEM((2, CHUNK), jnp.int32)] * NBUF
            + [pltpu.VMEM((CHUNK, F), jnp.float32)] * NBUF
            + [pltpu.VMEM_SHARED((N_PAD, F), jnp.float32)]
            + [pltpu.SemaphoreType.DMA] * (2 * NBUF)
        ),
    )(y, idx4)


def _bn_block(t, st, g, bt):
    inv_n = 1.0 / N
    mean = st[0:1, :] * inv_n
    var = st[1:2, :] * inv_n - mean * mean
    istd = lax.rsqrt(var + 1e-5)
    return (t - mean) * istd * g + bt


_spec_p0 = pl.BlockSpec((1, RB, F), lambda i: (0, i, 0))
_spec_p1 = pl.BlockSpec((1, RB, F), lambda i: (1, i, 0))
_spec_d0 = pl.BlockSpec((1, RB, 16), lambda i: (0, i, 0))
_spec_d1 = pl.BlockSpec((1, RB, 16), lambda i: (1, i, 0))
_spec_row = pl.BlockSpec((RB, F), lambda i: (i, 0))
_spec_w = pl.BlockSpec((F, F), lambda i: (0, 0))
_spec_vec = pl.BlockSpec((1, F), lambda i: (0, 0))
_spec_st = pl.BlockSpec((8, F), lambda i: (0, 0))
_spec_dis = pl.BlockSpec((RB, 1), lambda i: (i, 0))


def _dis_y1_body(d0_ref, d1_ref, xw_ref, dis_ref, y_ref):
    dis = lax.rsqrt(1.0 + d0_ref[0][:, :1] + d1_ref[0][:, :1])
    dis_ref[...] = dis
    y_ref[...] = dis * xw_ref[...]


def _tc_dis_y1(degp, xw1):
    return pl.pallas_call(
        _dis_y1_body,
        grid=(GRID,),
        in_specs=[_spec_p0, _spec_p1, _spec_row],
        out_specs=[_spec_dis, _spec_row],
        out_shape=[jax.ShapeDtypeStruct((N, 1), jnp.float32),
                   jax.ShapeDtypeStruct((N, F), jnp.float32)],
    )(degp, degp, xw1)


def _mm_body(h_ref, w_ref, y_ref):
    y_ref[...] = jnp.dot(h_ref[...], w_ref[...],
                         preferred_element_type=jnp.float32)


def _tc_matmul(h, W):
    return pl.pallas_call(
        _mm_body,
        grid=(GRID,),
        in_specs=[_spec_row, _spec_w],
        out_specs=_spec_row,
        out_shape=jax.ShapeDtypeStruct((N, F), jnp.float32),
    )(h, W)


def _mm_bn_body(t_ref, st_ref, g_ref, bt_ref, w_ref, dis_ref, y_ref):
    h = _bn_block(t_ref[...], st_ref[...], g_ref[...], bt_ref[...])
    y_ref[...] = dis_ref[...] * jnp.dot(h, w_ref[...],
                                        preferred_element_type=jnp.float32)


def _tc_matmul_bn(t, st, g, bt, W, dis):
    return pl.pallas_call(
        _mm_bn_body,
        grid=(GRID,),
        in_specs=[_spec_row, _spec_st, _spec_vec, _spec_vec, _spec_w,
                  _spec_dis],
        out_specs=_spec_row,
        out_shape=jax.ShapeDtypeStruct((N, F), jnp.float32),
    )(t, st, g, bt, W, dis)


def _mm3_body(x_ref, t1_ref, st1_ref, t2_ref, st2_ref, g_ref, bt_ref,
              w0_ref, w1_ref, w2_ref, dis_ref, y_ref):
    h1 = _bn_block(t1_ref[...], st1_ref[...], g_ref[...], bt_ref[...])
    h2 = _bn_block(t2_ref[...], st2_ref[...], g_ref[...], bt_ref[...])
    acc = jnp.dot(x_ref[...], w0_ref[...], preferred_element_type=jnp.float32)
    acc += jnp.dot(h1, w1_ref[...], preferred_element_type=jnp.float32)
    acc += jnp.dot(h2, w2_ref[...], preferred_element_type=jnp.float32)
    y_ref[...] = dis_ref[...] * acc


def _tc_matmul3_bn(x, t1, st1, t2, st2, g, bt, w0, w1, w2, dis):
    return pl.pallas_call(
        _mm3_body,
        grid=(GRID,),
        in_specs=[_spec_row, _spec_row, _spec_st, _spec_row, _spec_st,
                  _spec_vec, _spec_vec, _spec_w, _spec_w, _spec_w, _spec_dis],
        out_specs=_spec_row,
        out_shape=jax.ShapeDtypeStruct((N, F), jnp.float32),
    )(x, t1, st1, t2, st2, g, bt, w0, w1, w2, dis)


def _post_body(p0_ref, p1_ref, y_ref, dis_ref, b_ref, t_ref, st_ref):
    i = pl.program_id(0)
    pre = dis_ref[...] * (p0_ref[0] + p1_ref[0] + y_ref[...]) + b_ref[...]
    t = jnp.maximum(pre, 0.0)
    t_ref[...] = t
    ssum = jnp.sum(t, axis=0, keepdims=True)
    ssq = jnp.sum(t * t, axis=0, keepdims=True)
    st = jnp.concatenate([ssum, ssq, jnp.zeros((6, F), jnp.float32)], axis=0)

    @pl.when(i == 0)
    def _():
        st_ref[...] = st

    @pl.when(i != 0)
    def _():
        st_ref[...] += st


def _tc_post(p, y, dis, b):
    return pl.pallas_call(
        _post_body,
        grid=(GRID,),
        in_specs=[_spec_p0, _spec_p1, _spec_row, _spec_dis, _spec_vec],
        out_specs=[_spec_row, _spec_st],
        out_shape=[jax.ShapeDtypeStruct((N, F), jnp.float32),
                   jax.ShapeDtypeStruct((8, F), jnp.float32)],
    )(p, p, y, dis, b)


def _final_body(p0_ref, p1_ref, y_ref, dis_ref, b_ref, o_ref):
    pre = dis_ref[...] * (p0_ref[0] + p1_ref[0] + y_ref[...]) + b_ref[...]
    o_ref[...] = jnp.maximum(pre, 0.0)


def _tc_final(p, y, dis, b):
    return pl.pallas_call(
        _final_body,
        grid=(GRID,),
        in_specs=[_spec_p0, _spec_p1, _spec_row, _spec_dis, _spec_vec],
        out_specs=_spec_row,
        out_shape=jax.ShapeDtypeStruct((N, F), jnp.float32),
    )(p, p, y, dis, b)


def kernel(x, edge_index, W1, b1, W2, b2, gamma, beta, W_out, b_out):
    src = edge_index[0].astype(jnp.int32)
    dst = edge_index[1].astype(jnp.int32)
    e = src.shape[0]

    pad = E_PAD - e
    src_p = jnp.concatenate([src, jnp.zeros((pad,), jnp.int32)])
    dst_p = jnp.concatenate([dst, jnp.full((pad,), N, jnp.int32)])
    e0 = NS * CPW0 * CHUNK
    core0 = jnp.concatenate(
        [src_p[:e0].reshape(NS, CPW0, 1, CHUNK),
         dst_p[:e0].reshape(NS, CPW0, 1, CHUNK)], axis=2)
    core1 = jnp.concatenate(
        [src_p[e0:].reshape(NS, CPW1, 1, CHUNK),
         dst_p[e0:].reshape(NS, CPW1, 1, CHUNK)], axis=2)
    core1 = jnp.pad(core1, ((0, 0), (0, CPW0 - CPW1), (0, 0), (0, 0)))
    idx4 = jnp.concatenate([core0, core1], axis=0)

    pads = E_PADS - e
    dst_s = jnp.concatenate([dst, jnp.full((pads,), N, jnp.int32)])
    dst4 = dst_s.reshape(NW, CPWS, 1, CHUNK)

    b1r = b1.reshape(1, F)
    b2r = b2.reshape(1, F)
    bor = b_out.reshape(1, F)
    gr = gamma.reshape(1, F)
    btr = beta.reshape(1, F)

    degp = _sc_degrees(dst4)
    xw1 = _tc_matmul(x, W1)
    dis, y1 = _tc_dis_y1(degp, xw1)

    p1 = _sc_aggregate(y1, idx4)
    t1, st1 = _tc_post(p1, y1, dis, b1r)

    y2 = _tc_matmul_bn(t1, st1, gr, btr, W2, dis)
    p2 = _sc_aggregate(y2, idx4)
    t2, st2 = _tc_post(p2, y2, dis, b2r)

    y3 = _tc_matmul3_bn(x, t1, st1, t2, st2, gr, btr,
                        W_out[:F], W_out[F:2 * F], W_out[2 * F:], dis)
    p3 = _sc_aggregate(y3, idx4)
    return _tc_final(p3, y3, dis, bor)
